# Initial kernel scaffold; baseline (speedup 1.0000x reference)
import jax
import jax.numpy as jnp
from jax.experimental import pallas as pl


def kernel(inputs, gate_w, expert_w, expert_b):
    raise NotImplementedError("write your pallas kernel here")


# speedup vs baseline: 3.3074x; 3.3074x over previous
import jax
import jax.numpy as jnp
from jax.experimental import pallas as pl
from jax.experimental.pallas import tpu as pltpu

IN_FEATURES = 1024
OUT_FEATURES = 1024
N_EXPERTS = 8
K_TOP = 2
NEG_INF = float("-inf")


def _moe_dense_body(x_ref, gw_ref, w_ref, b_ref, o_ref, route_ref):
    e = pl.program_id(0)

    @pl.when(e == 0)
    def _compute_routing():
        logits = jax.lax.dot_general(
            x_ref[...], gw_ref[...],
            (((1,), (1,)), ((), ())),
            preferred_element_type=jnp.float32,
        )
        E = logits.shape[1]
        lane = jax.lax.broadcasted_iota(
            jnp.int32, logits.shape, 1).astype(jnp.float32)
        m1 = jnp.max(logits, axis=1, keepdims=True)
        i1 = jnp.min(jnp.where(logits == m1, lane, jnp.float32(E)), axis=1,
                     keepdims=True)
        masked = jnp.where(lane == i1, NEG_INF, logits)
        m2 = jnp.max(masked, axis=1, keepdims=True)
        i2 = jnp.min(jnp.where(masked == m2, lane, jnp.float32(E)), axis=1,
                     keepdims=True)
        z = jnp.exp(m2 - m1)
        denom = 1.0 + z
        w1 = 1.0 / denom
        w2 = z / denom
        route_ref[...] = jnp.concatenate([i1, w1, i2, w2], axis=1)

    i1 = route_ref[:, 0:1]
    w1 = route_ref[:, 1:2]
    i2 = route_ref[:, 2:3]
    w2 = route_ref[:, 3:4]
    ef = jnp.float32(1) * e
    w_e = jnp.where(i1 == ef, w1, 0.0) + jnp.where(i2 == ef, w2, 0.0)

    y = jax.lax.dot_general(
        x_ref[...], w_ref[0],
        (((1,), (1,)), ((), ())),
        preferred_element_type=jnp.float32,
    )
    contrib = w_e * (y + b_ref[0])

    @pl.when(e == 0)
    def _init():
        o_ref[...] = contrib

    @pl.when(e != 0)
    def _acc():
        o_ref[...] += contrib


def kernel(inputs, gate_w, expert_w, expert_b):
    B, S, D = inputs.shape
    E, F, _ = expert_w.shape
    x = inputs.reshape(S, D)

    out = pl.pallas_call(
        _moe_dense_body,
        grid=(E,),
        in_specs=[
            pl.BlockSpec((S, D), lambda e: (0, 0)),
            pl.BlockSpec((E, D), lambda e: (0, 0)),
            pl.BlockSpec((1, F, D), lambda e: (e, 0, 0)),
            pl.BlockSpec((1, 1, F), lambda e: (e, 0, 0)),
        ],
        out_specs=pl.BlockSpec((S, F), lambda e: (0, 0)),
        out_shape=jax.ShapeDtypeStruct((S, F), jnp.float32),
        scratch_shapes=[pltpu.VMEM((S, 4), jnp.float32)],
        compiler_params=pltpu.CompilerParams(
            dimension_semantics=("arbitrary",),
        ),
    )(x, gate_w, expert_w, expert_b.reshape(E, 1, F))
    return out.reshape(B, S, F)
